# Initial kernel scaffold; baseline (speedup 1.0000x reference)
import jax
import jax.numpy as jnp
from jax.experimental import pallas as pl


def kernel(trajectory, centers):
    raise NotImplementedError("write your pallas kernel here")


# speedup vs baseline: 19.1272x; 19.1272x over previous
import functools

import jax
import jax.numpy as jnp
from jax import lax
from jax.experimental import pallas as pl
from jax.experimental.pallas import tpu as pltpu
from jax.experimental.pallas import tpu_sc as plsc

B = 16384
K = 8192
D = 32
BT = 1024
BT2 = 2048

_NC = 2
_NS = 16
_NW = _NC * _NS
_BPW = B // _NW


SB = 256
LG = 128


def _split3(x):
    hi = x.astype(jnp.bfloat16)
    r = x - hi.astype(jnp.float32)
    lo = r.astype(jnp.bfloat16)
    lo2 = (r - lo.astype(jnp.float32)).astype(jnp.bfloat16)
    return hi, lo, lo2


def _top2_block(t_ref, c_ref, a1_ref, a2_ref, rhs_s):
    t = t_ref[...]
    inf = jnp.float32(jnp.inf)
    bigf = jnp.float32(65536.0)

    @pl.when(pl.program_id(0) == 0)
    def _build_rhs():
        cc = c_ref[...]
        cn = jnp.sum(cc * cc, axis=1, keepdims=True)
        cc_aug = jnp.concatenate([cc, cn], axis=1)
        c_hi, c_lo, c_lo2 = _split3(cc_aug)
        rhs_s[...] = jnp.concatenate(
            [c_hi, c_hi, c_hi, c_lo, c_lo2, c_lo], axis=1)

    t_aug = jnp.concatenate(
        [-2.0 * t, jnp.ones((BT, 1), jnp.float32)], axis=1)
    t_hi, t_lo, t_lo2 = _split3(t_aug)
    lhs = jnp.concatenate(
        [t_hi, t_lo, t_lo2, t_hi, t_hi, t_lo], axis=1)
    rhs = rhs_s[...]

    lane = lax.broadcasted_iota(jnp.int32, (SB, LG), 1).astype(jnp.float32)

    for sb in range(BT // SB):
        s = lax.dot_general(
            lhs[sb * SB:(sb + 1) * SB, :], rhs,
            (((1,), (1,)), ((), ())),
            preferred_element_type=jnp.float32,
        )
        m1 = s[:, 0:LG]
        i1 = jnp.zeros((SB, LG), jnp.float32)
        for g in range(1, K // LG):
            v = s[:, g * LG:(g + 1) * LG]
            b1 = v < m1
            i1 = jnp.where(b1, jnp.float32(g), i1)
            m1 = jnp.minimum(v, m1)
        flat1 = i1 * jnp.float32(LG) + lane
        gm1 = jnp.min(m1, axis=1, keepdims=True)
        gf1 = jnp.min(jnp.where(m1 == gm1, flat1, bigf), axis=1, keepdims=True)
        is_w = flat1 == gf1
        cm = jnp.where(is_w, inf, m1)
        gm2 = jnp.min(cm, axis=1, keepdims=True)
        gf2 = jnp.min(jnp.where(cm == gm2, flat1, bigf), axis=1, keepdims=True)
        a1_ref[pl.ds(sb * SB, SB), :] = gf1.astype(jnp.int32)
        a2_ref[pl.ds(sb * SB, SB), :] = gf2.astype(jnp.int32)


def _top2(trajectory, centers):
    return pl.pallas_call(
        _top2_block,
        grid=(B // BT,),
        in_specs=[
            pl.BlockSpec((BT, D), lambda b: (b, 0)),
            pl.BlockSpec((K, D), lambda b: (0, 0)),
        ],
        out_specs=[
            pl.BlockSpec((BT, 1), lambda b: (b, 0)),
            pl.BlockSpec((BT, 1), lambda b: (b, 0)),
        ],
        out_shape=[
            jax.ShapeDtypeStruct((B, 1), jnp.int32),
            jax.ShapeDtypeStruct((B, 1), jnp.int32),
        ],
        scratch_shapes=[
            pltpu.VMEM((K, 6 * (D + 1)), jnp.bfloat16),
        ],
        compiler_params=pltpu.CompilerParams(
            dimension_semantics=("arbitrary",),
        ),
    )(trajectory, centers)


def _sc_gather_body(c_hbm, a1_hbm, a2_hbm, g1_hbm, g2_hbm,
                    idx1_v, idx2_v, rows1_v, rows2_v, sem1, sem2):
    wid = lax.axis_index("s") * _NC + lax.axis_index("c")
    base = wid * _BPW
    pltpu.sync_copy(a1_hbm.at[pl.ds(base, _BPW)], idx1_v)
    pltpu.sync_copy(a2_hbm.at[pl.ds(base, _BPW)], idx2_v)
    cp1 = pltpu.async_copy(c_hbm.at[idx1_v], rows1_v, sem1)
    cp2 = pltpu.async_copy(c_hbm.at[idx2_v], rows2_v, sem2)
    cp1.wait()
    pltpu.sync_copy(rows1_v, g1_hbm.at[pl.ds(base, _BPW)])
    cp2.wait()
    pltpu.sync_copy(rows2_v, g2_hbm.at[pl.ds(base, _BPW)])


def _sc_gather(centers, a1, a2):
    mesh = plsc.VectorSubcoreMesh(core_axis_name="c", subcore_axis_name="s")
    fn = functools.partial(
        pl.kernel,
        mesh=mesh,
        compiler_params=pltpu.CompilerParams(use_tc_tiling_on_sc=False),
        out_type=[
            jax.ShapeDtypeStruct((B, D), jnp.float32),
            jax.ShapeDtypeStruct((B, D), jnp.float32),
        ],
        scratch_types=[
            pltpu.VMEM((_BPW,), jnp.int32),
            pltpu.VMEM((_BPW,), jnp.int32),
            pltpu.VMEM((_BPW, D), jnp.float32),
            pltpu.VMEM((_BPW, D), jnp.float32),
            pltpu.SemaphoreType.DMA,
            pltpu.SemaphoreType.DMA,
        ],
    )(_sc_gather_body)
    return fn(centers, a1, a2)


def _tree_sum_lanes(x):
    w = x.shape[1]
    while w > 1:
        h = w // 2
        x = x[:, :h] + x[:, h:]
        w = h
    return x


def _select_block(t_ref, g1_ref, g2_ref, a1_ref, a2_ref, idx_ref, off_ref):
    t = t_ref[...]
    g1 = g1_ref[...]
    g2 = g2_ref[...]
    a1 = a1_ref[...]
    a2 = a2_ref[...]
    e1 = t - g1
    e2 = t - g2
    d1 = jnp.sqrt(_tree_sum_lanes(e1 * e1))
    d2 = jnp.sqrt(_tree_sum_lanes(e2 * e2))
    pick1 = (d1 < d2) | ((d1 == d2) & (a1 < a2))
    idx_ref[...] = jnp.where(pick1, a1, a2)
    off_ref[...] = jnp.where(pick1, e1, e2)


def _select(trajectory, g1, g2, a1, a2):
    return pl.pallas_call(
        _select_block,
        grid=(B // BT2,),
        in_specs=[
            pl.BlockSpec((BT2, D), lambda i: (i, 0)),
            pl.BlockSpec((BT2, D), lambda i: (i, 0)),
            pl.BlockSpec((BT2, D), lambda i: (i, 0)),
            pl.BlockSpec((BT2, 1), lambda i: (i, 0)),
            pl.BlockSpec((BT2, 1), lambda i: (i, 0)),
        ],
        out_specs=[
            pl.BlockSpec((BT2, 1), lambda i: (i, 0)),
            pl.BlockSpec((BT2, D), lambda i: (i, 0)),
        ],
        out_shape=[
            jax.ShapeDtypeStruct((B, 1), jnp.int32),
            jax.ShapeDtypeStruct((B, D), jnp.float32),
        ],
    )(trajectory, g1, g2, a1, a2)


@jax.jit
def kernel(trajectory, centers):
    a1, a2 = _top2(trajectory, centers)
    g1, g2 = _sc_gather(centers, a1.reshape(B), a2.reshape(B))
    idx, offset = _select(trajectory, g1, g2, a1, a2)
    return idx.reshape(B), offset
